# baseline (speedup 1.0000x reference)
import jax
import jax.numpy as jnp
from jax.experimental import pallas as pl
from jax.experimental.pallas import tpu as pltpu

NUM_CLASS = 1000
ALPHA = 0.25
GAMMA = 2.0
EPS = 1e-10

ROWS = 32768
HALF = ROWS // 2
BLOCK_R = 2048
N_BLOCKS = HALF // BLOCK_R


def _logpt(x, t):
    row_max = jnp.max(x, axis=1, keepdims=True)
    sumexp = jnp.sum(jnp.exp(x - row_max), axis=1, keepdims=True)
    cols = jax.lax.broadcasted_iota(jnp.int32, x.shape, 1)
    xt = jnp.sum(jnp.where(cols == t, x, 0.0), axis=1, keepdims=True)
    pt = jnp.exp(xt - row_max) / sumexp
    return jnp.log(pt + EPS)


def _loss_kernel(xlo_ref, xhi_ref, tlo_ref, thi_ref, anc_ref, pos_ref,
                 out_ref):
    sim = jnp.sum(anc_ref[...] * pos_ref[...], axis=1, keepdims=True)
    omp = 1.0 - sim
    weight = -ALPHA * omp * omp
    lp = _logpt(xlo_ref[...], tlo_ref[...]) + _logpt(xhi_ref[...], thi_ref[...])
    out_ref[...] = jnp.sum(weight * lp).reshape(1, 1, 1)


@jax.jit
def kernel(descriptors, input, target):
    tgt2d = target.reshape(ROWS, 1)
    partials = pl.pallas_call(
        _loss_kernel,
        grid=(N_BLOCKS,),
        in_specs=[
            pl.BlockSpec((BLOCK_R, NUM_CLASS), lambda i: (i, 0)),
            pl.BlockSpec((BLOCK_R, NUM_CLASS), lambda i: (i + N_BLOCKS, 0)),
            pl.BlockSpec((BLOCK_R, 1), lambda i: (i, 0)),
            pl.BlockSpec((BLOCK_R, 1), lambda i: (i + N_BLOCKS, 0)),
            pl.BlockSpec((BLOCK_R, 128), lambda i: (i, 0)),
            pl.BlockSpec((BLOCK_R, 128), lambda i: (i + N_BLOCKS, 0)),
        ],
        out_specs=pl.BlockSpec((1, 1, 1), lambda i: (i, 0, 0)),
        out_shape=jax.ShapeDtypeStruct((N_BLOCKS, 1, 1), jnp.float32),
        compiler_params=pltpu.CompilerParams(
            dimension_semantics=("parallel",)),
    )(input, input, tgt2d, tgt2d, descriptors, descriptors)
    return jnp.sum(partials) / ROWS
